# baseline (speedup 1.0000x reference)
import functools

import jax
import jax.numpy as jnp
from jax import lax
from jax.experimental import pallas as pl
from jax.experimental.pallas import tpu as pltpu
from jax.experimental.pallas import tpu_sc as plsc

N = 10000
E = 320000
D = 128
NODE_MASK_NUM = 1000
EDGE_DROP = 32000
EKEEP = E - EDGE_DROP

NC = 2
NS = 16
NW = NC * NS

NPAD = 10240
RB = 1024
NB = NPAD // RB
CH = 96
NJT = 192
NCHUNKS = NS * NJT
EPAD = NCHUNKS * CH
NJD0 = 120
NJD1 = NCHUNKS // NS - NJD0
ROWS_PER_TILE = NPAD // NS
NR = NPAD // 128
_PIECES = [CH] * (ROWS_PER_TILE // CH) + (
    [ROWS_PER_TILE % CH] if ROWS_PER_TILE % CH else [])

_HIGH = jax.lax.Precision.HIGHEST
_SC_PARAMS = pltpu.CompilerParams(needs_layout_passes=False,
                                  use_tc_tiling_on_sc=False)


def _mesh():
    return plsc.VectorSubcoreMesh(core_axis_name="c", subcore_axis_name="s",
                                  num_cores=NC, num_subcores=NS)


def _zero_2d(ref, nrows, ncols):
    z = jnp.zeros((16,), ref.dtype)

    @pl.loop(0, nrows)
    def _(r):
        for cc in range(ncols // 16):
            ref[r, pl.ds(cc * 16, 16)] = z


def _unpack(pkbuf, j, sb, db):
    for i in range(CH // 16):
        p = pkbuf[j, pl.ds(i * 16, 16)]
        sb[pl.ds(i * 16, 16)] = jnp.bitwise_and(p, 0xFFFF)
        db[pl.ds(i * 16, 16)] = jnp.right_shift(p, 16)


def _sc_degree(pk2):
    @functools.partial(
        pl.kernel,
        out_type=jax.ShapeDtypeStruct((NW, NR, 128), jnp.float32),
        mesh=_mesh(),
        compiler_params=_SC_PARAMS,
        scratch_types=[
            pltpu.VMEM((NJD0, CH), jnp.int32),
            pltpu.VMEM((NR, 128), jnp.float32),
        ],
    )
    def k(pk_hbm, degp_out, pkbuf, acc):
        c = lax.axis_index("c")
        s = lax.axis_index("s")
        wid = s * NC + c
        lo = jnp.where(c == 0, s * NJD0, NS * NJD0 + s * NJD1)
        njc = jnp.where(c == 0, NJD0, NJD1)

        @pl.when(c == 0)
        def _():
            pltpu.sync_copy(pk_hbm.at[pl.ds(lo, NJD0)], pkbuf)

        @pl.when(c == 1)
        def _():
            pltpu.sync_copy(pk_hbm.at[pl.ds(lo, NJD1)],
                            pkbuf.at[pl.ds(0, NJD1)])

        _zero_2d(acc, NR, 128)
        ones = jnp.ones((16,), jnp.float32)

        @pl.loop(0, njc)
        def _(j):
            for i in range(CH // 16):
                idx = jnp.right_shift(pkbuf[j, pl.ds(i * 16, 16)], 16)
                plsc.addupdate_scatter(
                    acc, [jnp.right_shift(idx, 7), jnp.bitwise_and(idx, 127)],
                    ones)

        pltpu.sync_copy(acc, degp_out.at[wid])

    return k(pk2)


def _tc_prepare_body(degp_ref, x_ref, w1_ref, y_ref, dinv_ref):
    i = pl.program_id(0)
    ones_nw = jnp.ones((NW, 1), jnp.float32)
    deg = lax.dot_general(degp_ref[...], ones_nw,
                          (((0,), (0,)), ((), ())),
                          preferred_element_type=jnp.float32)
    dinv = lax.rsqrt(deg + 1.0)
    rows = i * RB + lax.broadcasted_iota(jnp.int32, (RB, 1), 0)
    xm = jnp.where(rows < NODE_MASK_NUM, 0.0, x_ref[...])
    xw = jnp.dot(xm, w1_ref[...], preferred_element_type=jnp.float32,
                 precision=_HIGH)
    y_ref[...] = xw * dinv
    dinv_ref[...] = dinv


def _tc_prepare(degp, x_pad, W1):
    return pl.pallas_call(
        _tc_prepare_body,
        grid=(NB,),
        in_specs=[
            pl.BlockSpec((NW, RB), lambda i: (0, i)),
            pl.BlockSpec((RB, D), lambda i: (i, 0)),
            pl.BlockSpec((D, D), lambda i: (0, 0)),
        ],
        out_specs=[
            pl.BlockSpec((RB, D), lambda i: (i, 0)),
            pl.BlockSpec((RB, 1), lambda i: (i, 0)),
        ],
        out_shape=[
            jax.ShapeDtypeStruct((NPAD, D), jnp.float32),
            jax.ShapeDtypeStruct((NPAD, 1), jnp.float32),
        ],
    )(degp, x_pad, W1)


def _sc_scatter(y, dinv, pk2):
    @functools.partial(
        pl.kernel,
        out_type=[
            jax.ShapeDtypeStruct((NPAD, D), jnp.float32),
            jax.ShapeDtypeStruct((NPAD,), jnp.float32),
        ],
        mesh=_mesh(),
        compiler_params=_SC_PARAMS,
        scratch_types=[
            pltpu.VMEM_SHARED((NPAD, D), jnp.float32),
            pltpu.VMEM_SHARED((NPAD,), jnp.float32),
            pltpu.VMEM_SHARED((NPAD,), jnp.float32),
            pltpu.VMEM((NJT, CH), jnp.int32),
            pltpu.VMEM((CH, D), jnp.float32),
            pltpu.VMEM((CH, D), jnp.float32),
            pltpu.VMEM((CH,), jnp.int32),
            pltpu.VMEM((CH,), jnp.int32),
            pltpu.VMEM((CH,), jnp.int32),
            pltpu.VMEM((CH,), jnp.int32),
            pltpu.VMEM((CH,), jnp.float32),
            pltpu.VMEM((ROWS_PER_TILE,), jnp.float32),
            pltpu.SemaphoreType.DMA,
            pltpu.SemaphoreType.DMA,
        ],
    )
    def k(y_hbm, dinv_hbm, pk_hbm, agg_out, sp_out,
          agg_sh, dinv_sh, s_sh, pkbuf, rows0, rows1,
          sidx0, sidx1, didx0, didx1, vbuf, temp, sem0, sem1):
        c = lax.axis_index("c")
        s = lax.axis_index("s")
        r0 = s * ROWS_PER_TILE
        lo = s * NJT
        pltpu.sync_copy(pk_hbm.at[pl.ds(lo, NJT)], pkbuf)

        @pl.when(c == 0)
        def _():
            _zero_2d(rows0, CH, D)
            base = 0
            for sz in _PIECES:
                pltpu.sync_copy(rows0.at[pl.ds(0, sz)],
                                agg_sh.at[pl.ds(r0 + base, sz)])
                base += sz
            plsc.subcore_barrier()

            _unpack(pkbuf, 0, sidx0, didx0)
            pltpu.async_copy(y_hbm.at[sidx0], rows0, sem0)
            _unpack(pkbuf, 1, sidx1, didx1)
            pltpu.async_copy(y_hbm.at[sidx1], rows1, sem1)

            @pl.loop(0, NJT // 2)
            def _(t):
                j = t * 2
                pltpu.make_async_copy(y_hbm.at[sidx0], rows0, sem0).wait()
                pltpu.sync_copy(rows0, agg_sh.at[didx0], add=True)

                @pl.when(t < NJT // 2 - 1)
                def _():
                    _unpack(pkbuf, j + 2, sidx0, didx0)
                    pltpu.async_copy(y_hbm.at[sidx0], rows0, sem0)

                pltpu.make_async_copy(y_hbm.at[sidx1], rows1, sem1).wait()
                pltpu.sync_copy(rows1, agg_sh.at[didx1], add=True)

                @pl.when(t < NJT // 2 - 1)
                def _():
                    _unpack(pkbuf, j + 3, sidx1, didx1)
                    pltpu.async_copy(y_hbm.at[sidx1], rows1, sem1)

            plsc.subcore_barrier()
            base = 0
            for sz in _PIECES:
                pltpu.sync_copy(agg_sh.at[pl.ds(r0 + base, sz)],
                                rows0.at[pl.ds(0, sz)])
                pltpu.sync_copy(rows0.at[pl.ds(0, sz)],
                                agg_out.at[pl.ds(r0 + base, sz)])
                base += sz

        @pl.when(c == 1)
        def _():
            pltpu.sync_copy(dinv_hbm.at[pl.ds(r0, ROWS_PER_TILE)], temp)
            pltpu.sync_copy(temp, dinv_sh.at[pl.ds(r0, ROWS_PER_TILE)])
            z16 = jnp.zeros((16,), jnp.float32)

            @pl.loop(0, ROWS_PER_TILE // 16)
            def _(r):
                temp[pl.ds(r * 16, 16)] = z16

            pltpu.sync_copy(temp, s_sh.at[pl.ds(r0, ROWS_PER_TILE)])
            plsc.subcore_barrier()

            @pl.loop(0, NJT)
            def _(j):
                _unpack(pkbuf, j, sidx0, didx0)
                pltpu.sync_copy(dinv_sh.at[didx0], vbuf)
                pltpu.sync_copy(vbuf, s_sh.at[sidx0], add=True)

            plsc.subcore_barrier()
            pltpu.sync_copy(s_sh.at[pl.ds(r0, ROWS_PER_TILE)], temp)
            pltpu.sync_copy(temp, sp_out.at[pl.ds(r0, ROWS_PER_TILE)])

    return k(y, dinv, pk2)


def _tc_finish_body(agg_ref, sp_ref, y_ref, dinv_ref, b1_ref, w2_ref, b2_ref,
                    out_ref, acc_ref):
    i = pl.program_id(0)
    agg = agg_ref[...] + y_ref[...]
    dinv = dinv_ref[...]
    h1 = jnp.maximum(agg * dinv + b1_ref[...], 0.0)
    s_col = sp_ref[...]
    c_col = dinv * s_col + dinv * dinv
    rows = i * RB + lax.broadcasted_iota(jnp.int32, (RB, 1), 0)
    c_col = jnp.where(rows < N, c_col, 0.0)
    part = lax.dot_general(c_col, h1, (((0,), (0,)), ((), ())),
                           precision=_HIGH,
                           preferred_element_type=jnp.float32)

    @pl.when(i == 0)
    def _():
        acc_ref[...] = part

    @pl.when(i > 0)
    def _():
        acc_ref[...] = acc_ref[...] + part

    @pl.when(i == NB - 1)
    def _():
        r = acc_ref[...] * (1.0 / N)
        out_ref[...] = jnp.tanh(
            jnp.dot(r, w2_ref[...], preferred_element_type=jnp.float32,
                    precision=_HIGH) + b2_ref[...])


def _tc_finish(agg, sp, y, dinv, b1r, W2, b2r):
    return pl.pallas_call(
        _tc_finish_body,
        grid=(NB,),
        in_specs=[
            pl.BlockSpec((RB, D), lambda i: (i, 0)),
            pl.BlockSpec((RB, 1), lambda i: (i, 0)),
            pl.BlockSpec((RB, D), lambda i: (i, 0)),
            pl.BlockSpec((RB, 1), lambda i: (i, 0)),
            pl.BlockSpec((1, D), lambda i: (0, 0)),
            pl.BlockSpec((D, D), lambda i: (0, 0)),
            pl.BlockSpec((1, D), lambda i: (0, 0)),
        ],
        out_specs=pl.BlockSpec((1, D), lambda i: (0, 0)),
        out_shape=jax.ShapeDtypeStruct((1, D), jnp.float32),
        scratch_shapes=[pltpu.VMEM((1, D), jnp.float32)],
    )(agg, sp, y, dinv, b1r, W2, b2r)


def kernel(x, edge_index, W1, b1, W2, b2):
    src = edge_index[0, EDGE_DROP:].astype(jnp.int32)
    dst = edge_index[1, EDGE_DROP:].astype(jnp.int32)
    pad = jnp.full((EPAD - EKEEP,), N, jnp.int32)
    pk2 = (jnp.concatenate([src, pad])
           | (jnp.concatenate([dst, pad]) << 16)).reshape(NCHUNKS, CH)
    x_pad = jnp.concatenate(
        [x, jnp.zeros((NPAD - N, D), jnp.float32)], axis=0)

    degp = _sc_degree(pk2).reshape(NW, NPAD)
    y, dinv = _tc_prepare(degp, x_pad, W1)
    agg, sp = _sc_scatter(y, dinv.reshape(NPAD), pk2)
    out = _tc_finish(agg, sp.reshape(NPAD, 1), y, dinv,
                     b1.reshape(1, D), W2, b2.reshape(1, D))
    return out


# speedup vs baseline: 1.0172x; 1.0172x over previous
import functools

import jax
import jax.numpy as jnp
from jax import lax
from jax.experimental import pallas as pl
from jax.experimental.pallas import tpu as pltpu
from jax.experimental.pallas import tpu_sc as plsc

N = 10000
E = 320000
D = 128
NODE_MASK_NUM = 1000
EDGE_DROP = 32000
EKEEP = E - EDGE_DROP

NC = 2
NS = 16
NW = NC * NS

NPAD = 10240
RB = 1024
NB = NPAD // RB
CH = 64
NJT = 288
NCHUNKS = NS * NJT
EPAD = NCHUNKS * CH
NJD0 = 180
NJD1 = NCHUNKS // NS - NJD0
ROWS_PER_TILE = NPAD // NS
NR = NPAD // 128
_PIECES = [CH] * (ROWS_PER_TILE // CH) + (
    [ROWS_PER_TILE % CH] if ROWS_PER_TILE % CH else [])

_HIGH = jax.lax.Precision.HIGHEST
_SC_PARAMS = pltpu.CompilerParams(needs_layout_passes=False,
                                  use_tc_tiling_on_sc=False)


def _mesh():
    return plsc.VectorSubcoreMesh(core_axis_name="c", subcore_axis_name="s",
                                  num_cores=NC, num_subcores=NS)


def _zero_2d(ref, nrows, ncols):
    z = jnp.zeros((16,), ref.dtype)

    @pl.loop(0, nrows)
    def _(r):
        for cc in range(ncols // 16):
            ref[r, pl.ds(cc * 16, 16)] = z


def _unpack(pkbuf, j, sb, db):
    for i in range(CH // 16):
        p = pkbuf[j, pl.ds(i * 16, 16)]
        sb[pl.ds(i * 16, 16)] = jnp.bitwise_and(p, 0xFFFF)
        db[pl.ds(i * 16, 16)] = jnp.right_shift(p, 16)


def _sc_degree(pk2):
    @functools.partial(
        pl.kernel,
        out_type=jax.ShapeDtypeStruct((NW, NR, 128), jnp.float32),
        mesh=_mesh(),
        compiler_params=_SC_PARAMS,
        scratch_types=[
            pltpu.VMEM((NJD0, CH), jnp.int32),
            pltpu.VMEM((NR, 128), jnp.float32),
        ],
    )
    def k(pk_hbm, degp_out, pkbuf, acc):
        c = lax.axis_index("c")
        s = lax.axis_index("s")
        wid = s * NC + c
        lo = jnp.where(c == 0, s * NJD0, NS * NJD0 + s * NJD1)
        njc = jnp.where(c == 0, NJD0, NJD1)

        @pl.when(c == 0)
        def _():
            pltpu.sync_copy(pk_hbm.at[pl.ds(lo, NJD0)], pkbuf)

        @pl.when(c == 1)
        def _():
            pltpu.sync_copy(pk_hbm.at[pl.ds(lo, NJD1)],
                            pkbuf.at[pl.ds(0, NJD1)])

        _zero_2d(acc, NR, 128)
        ones = jnp.ones((16,), jnp.float32)

        @pl.loop(0, njc)
        def _(j):
            for i in range(CH // 16):
                idx = jnp.right_shift(pkbuf[j, pl.ds(i * 16, 16)], 16)
                plsc.addupdate_scatter(
                    acc, [jnp.right_shift(idx, 7), jnp.bitwise_and(idx, 127)],
                    ones)

        pltpu.sync_copy(acc, degp_out.at[wid])

    return k(pk2)


def _tc_prepare_body(degp_ref, x_ref, w1_ref, y_ref, dinv_ref):
    i = pl.program_id(0)
    ones_nw = jnp.ones((NW, 1), jnp.float32)
    deg = lax.dot_general(degp_ref[...], ones_nw,
                          (((0,), (0,)), ((), ())),
                          preferred_element_type=jnp.float32)
    dinv = lax.rsqrt(deg + 1.0)
    rows = i * RB + lax.broadcasted_iota(jnp.int32, (RB, 1), 0)
    xm = jnp.where(rows < NODE_MASK_NUM, 0.0, x_ref[...])
    xw = jnp.dot(xm, w1_ref[...], preferred_element_type=jnp.float32,
                 precision=_HIGH)
    y_ref[...] = xw * dinv
    dinv_ref[...] = dinv


def _tc_prepare(degp, x_pad, W1):
    return pl.pallas_call(
        _tc_prepare_body,
        grid=(NB,),
        in_specs=[
            pl.BlockSpec((NW, RB), lambda i: (0, i)),
            pl.BlockSpec((RB, D), lambda i: (i, 0)),
            pl.BlockSpec((D, D), lambda i: (0, 0)),
        ],
        out_specs=[
            pl.BlockSpec((RB, D), lambda i: (i, 0)),
            pl.BlockSpec((RB, 1), lambda i: (i, 0)),
        ],
        out_shape=[
            jax.ShapeDtypeStruct((NPAD, D), jnp.float32),
            jax.ShapeDtypeStruct((NPAD, 1), jnp.float32),
        ],
    )(degp, x_pad, W1)


def _sc_scatter(y, dinv, pk2):
    @functools.partial(
        pl.kernel,
        out_type=[
            jax.ShapeDtypeStruct((NPAD, D), jnp.float32),
            jax.ShapeDtypeStruct((NPAD,), jnp.float32),
        ],
        mesh=_mesh(),
        compiler_params=_SC_PARAMS,
        scratch_types=[
            pltpu.VMEM_SHARED((NPAD, D), jnp.float32),
            pltpu.VMEM_SHARED((NPAD,), jnp.float32),
            pltpu.VMEM_SHARED((NPAD,), jnp.float32),
            pltpu.VMEM((NJT, CH), jnp.int32),
            [pltpu.VMEM((CH, D), jnp.float32)] * 3,
            [pltpu.VMEM((CH,), jnp.int32)] * 3,
            [pltpu.VMEM((CH,), jnp.int32)] * 3,
            pltpu.VMEM((CH,), jnp.float32),
            pltpu.VMEM((ROWS_PER_TILE,), jnp.float32),
            [pltpu.SemaphoreType.DMA] * 3,
            [pltpu.SemaphoreType.DMA] * 3,
        ],
    )
    def k(y_hbm, dinv_hbm, pk_hbm, agg_out, sp_out,
          agg_sh, dinv_sh, s_sh, pkbuf, rows, sidx, didx, vbuf, temp,
          semg, sems):
        c = lax.axis_index("c")
        s = lax.axis_index("s")
        r0 = s * ROWS_PER_TILE
        lo = s * NJT
        pltpu.sync_copy(pk_hbm.at[pl.ds(lo, NJT)], pkbuf)

        def _fire_gather(m, b):
            _unpack(pkbuf, m, sidx[b], didx[b])
            pltpu.async_copy(y_hbm.at[sidx[b]], rows[b], semg[b])

        def _wait_gather(b):
            pltpu.make_async_copy(y_hbm.at[sidx[b]], rows[b], semg[b]).wait()

        def _fire_scatter(b):
            pltpu.async_copy(rows[b], agg_sh.at[didx[b]], sems[b], add=True)

        def _wait_scatter(b):
            pltpu.make_async_copy(rows[b], agg_sh.at[didx[b]], sems[b]).wait()

        @pl.when(c == 0)
        def _():
            _zero_2d(rows[0], CH, D)
            base = 0
            for sz in _PIECES:
                pltpu.sync_copy(rows[0].at[pl.ds(0, sz)],
                                agg_sh.at[pl.ds(r0 + base, sz)])
                base += sz
            plsc.subcore_barrier()

            _fire_gather(0, 0)
            _fire_gather(1, 1)
            NIT = NJT // 3

            @pl.loop(0, NIT)
            def _(t):
                j = t * 3
                for b in range(3):
                    nb = (b + 2) % 3
                    _wait_gather(b)
                    _fire_scatter(b)
                    if b == 0:
                        @pl.when(t > 0)
                        def _():
                            _wait_scatter(nb)

                        _fire_gather(j + 2, nb)
                    else:
                        @pl.when(t < NIT - 1)
                        def _():
                            _wait_scatter(nb)
                            _fire_gather(j + b + 2, nb)

            for b in range(3):
                _wait_scatter(b)
            plsc.subcore_barrier()
            base = 0
            for sz in _PIECES:
                pltpu.sync_copy(agg_sh.at[pl.ds(r0 + base, sz)],
                                rows[0].at[pl.ds(0, sz)])
                pltpu.sync_copy(rows[0].at[pl.ds(0, sz)],
                                agg_out.at[pl.ds(r0 + base, sz)])
                base += sz

        @pl.when(c == 1)
        def _():
            pltpu.sync_copy(dinv_hbm.at[pl.ds(r0, ROWS_PER_TILE)], temp)
            pltpu.sync_copy(temp, dinv_sh.at[pl.ds(r0, ROWS_PER_TILE)])
            z16 = jnp.zeros((16,), jnp.float32)

            @pl.loop(0, ROWS_PER_TILE // 16)
            def _(r):
                temp[pl.ds(r * 16, 16)] = z16

            pltpu.sync_copy(temp, s_sh.at[pl.ds(r0, ROWS_PER_TILE)])
            plsc.subcore_barrier()

            @pl.loop(0, NJT)
            def _(j):
                _unpack(pkbuf, j, sidx[0], didx[0])
                pltpu.sync_copy(dinv_sh.at[didx[0]], vbuf)
                pltpu.sync_copy(vbuf, s_sh.at[sidx[0]], add=True)

            plsc.subcore_barrier()
            pltpu.sync_copy(s_sh.at[pl.ds(r0, ROWS_PER_TILE)], temp)
            pltpu.sync_copy(temp, sp_out.at[pl.ds(r0, ROWS_PER_TILE)])

    return k(y, dinv, pk2)


def _tc_finish_body(agg_ref, sp_ref, y_ref, dinv_ref, b1_ref, w2_ref, b2_ref,
                    out_ref, acc_ref):
    i = pl.program_id(0)
    agg = agg_ref[...] + y_ref[...]
    dinv = dinv_ref[...]
    h1 = jnp.maximum(agg * dinv + b1_ref[...], 0.0)
    s_col = sp_ref[...]
    c_col = dinv * s_col + dinv * dinv
    rows = i * RB + lax.broadcasted_iota(jnp.int32, (RB, 1), 0)
    c_col = jnp.where(rows < N, c_col, 0.0)
    part = lax.dot_general(c_col, h1, (((0,), (0,)), ((), ())),
                           precision=_HIGH,
                           preferred_element_type=jnp.float32)

    @pl.when(i == 0)
    def _():
        acc_ref[...] = part

    @pl.when(i > 0)
    def _():
        acc_ref[...] = acc_ref[...] + part

    @pl.when(i == NB - 1)
    def _():
        r = acc_ref[...] * (1.0 / N)
        out_ref[...] = jnp.tanh(
            jnp.dot(r, w2_ref[...], preferred_element_type=jnp.float32,
                    precision=_HIGH) + b2_ref[...])


def _tc_finish(agg, sp, y, dinv, b1r, W2, b2r):
    return pl.pallas_call(
        _tc_finish_body,
        grid=(NB,),
        in_specs=[
            pl.BlockSpec((RB, D), lambda i: (i, 0)),
            pl.BlockSpec((RB, 1), lambda i: (i, 0)),
            pl.BlockSpec((RB, D), lambda i: (i, 0)),
            pl.BlockSpec((RB, 1), lambda i: (i, 0)),
            pl.BlockSpec((1, D), lambda i: (0, 0)),
            pl.BlockSpec((D, D), lambda i: (0, 0)),
            pl.BlockSpec((1, D), lambda i: (0, 0)),
        ],
        out_specs=pl.BlockSpec((1, D), lambda i: (0, 0)),
        out_shape=jax.ShapeDtypeStruct((1, D), jnp.float32),
        scratch_shapes=[pltpu.VMEM((1, D), jnp.float32)],
    )(agg, sp, y, dinv, b1r, W2, b2r)


def kernel(x, edge_index, W1, b1, W2, b2):
    src = edge_index[0, EDGE_DROP:].astype(jnp.int32)
    dst = edge_index[1, EDGE_DROP:].astype(jnp.int32)
    pad = jnp.full((EPAD - EKEEP,), N, jnp.int32)
    pk2 = (jnp.concatenate([src, pad])
           | (jnp.concatenate([dst, pad]) << 16)).reshape(NCHUNKS, CH)
    x_pad = jnp.concatenate(
        [x, jnp.zeros((NPAD - N, D), jnp.float32)], axis=0)

    degp = _sc_degree(pk2).reshape(NW, NPAD)
    y, dinv = _tc_prepare(degp, x_pad, W1)
    agg, sp = _sc_scatter(y, dinv.reshape(NPAD), pk2)
    out = _tc_finish(agg, sp.reshape(NPAD, 1), y, dinv,
                     b1.reshape(1, D), W2, b2.reshape(1, D))
    return out


# speedup vs baseline: 2.7555x; 2.7090x over previous
import functools

import jax
import jax.numpy as jnp
from jax import lax
from jax.experimental import pallas as pl
from jax.experimental.pallas import tpu as pltpu
from jax.experimental.pallas import tpu_sc as plsc

N = 10000
E = 320000
D = 128
NODE_MASK_NUM = 1000
EDGE_DROP = 32000
EKEEP = E - EDGE_DROP

NC = 2
NS = 16
NW = NC * NS

NPAD = 10240
RB = 1024
NB = NPAD // RB
CH = 64
TPC = 288
NCHUNKS = NS * TPC
EPAD = NCHUNKS * CH
NJ0 = 144
NJ1 = TPC - NJ0
NJD0 = 144
NJD1 = TPC - NJD0
ROWS_PER_TILE = NPAD // NS
NR = NPAD // 128

_HIGH = jax.lax.Precision.HIGHEST
_SC_PARAMS = pltpu.CompilerParams(needs_layout_passes=False,
                                  use_tc_tiling_on_sc=False)


def _mesh():
    return plsc.VectorSubcoreMesh(core_axis_name="c", subcore_axis_name="s",
                                  num_cores=NC, num_subcores=NS)


def _zero_2d(ref, nrows, ncols):
    z = jnp.zeros((16,), ref.dtype)

    @pl.loop(0, nrows)
    def _(r):
        for cc in range(ncols // 16):
            ref[r, pl.ds(cc * 16, 16)] = z


def _sc_degree(dst2):
    @functools.partial(
        pl.kernel,
        out_type=jax.ShapeDtypeStruct((NW, NR, 128), jnp.float32),
        mesh=_mesh(),
        compiler_params=_SC_PARAMS,
        scratch_types=[
            pltpu.VMEM((NJD0, CH), jnp.int32),
            pltpu.VMEM((NR, 128), jnp.float32),
        ],
    )
    def k(dst_hbm, degp_out, dstbuf, acc):
        c = lax.axis_index("c")
        s = lax.axis_index("s")
        wid = s * NC + c
        lo = jnp.where(c == 0, s * NJD0, NS * NJD0 + s * NJD1)
        njc = jnp.where(c == 0, NJD0, NJD1)

        @pl.when(c == 0)
        def _():
            pltpu.sync_copy(dst_hbm.at[pl.ds(lo, NJD0)], dstbuf)

        @pl.when(c == 1)
        def _():
            pltpu.sync_copy(dst_hbm.at[pl.ds(lo, NJD1)],
                            dstbuf.at[pl.ds(0, NJD1)])

        _zero_2d(acc, NR, 128)
        ones = jnp.ones((16,), jnp.float32)

        @pl.loop(0, njc)
        def _(j):
            for i in range(CH // 16):
                idx = dstbuf[j, pl.ds(i * 16, 16)]
                plsc.addupdate_scatter(
                    acc, [jnp.right_shift(idx, 7), jnp.bitwise_and(idx, 127)],
                    ones)

        pltpu.sync_copy(acc, degp_out.at[wid])

    return k(dst2)


def _tc_prepare_body(degp_ref, x_ref, w1_ref, y_ref, dinv_ref):
    i = pl.program_id(0)
    ones_nw = jnp.ones((NW, 1), jnp.float32)
    deg = lax.dot_general(degp_ref[...], ones_nw,
                          (((0,), (0,)), ((), ())),
                          preferred_element_type=jnp.float32)
    dinv = lax.rsqrt(deg + 1.0)
    rows = i * RB + lax.broadcasted_iota(jnp.int32, (RB, 1), 0)
    xm = jnp.where(rows < NODE_MASK_NUM, 0.0, x_ref[...])
    xw = jnp.dot(xm, w1_ref[...], preferred_element_type=jnp.float32,
                 precision=_HIGH)
    y_ref[...] = xw * dinv
    dinv_ref[...] = dinv


def _tc_prepare(degp, x_pad, W1):
    return pl.pallas_call(
        _tc_prepare_body,
        grid=(NB,),
        in_specs=[
            pl.BlockSpec((NW, RB), lambda i: (0, i)),
            pl.BlockSpec((RB, D), lambda i: (i, 0)),
            pl.BlockSpec((D, D), lambda i: (0, 0)),
        ],
        out_specs=[
            pl.BlockSpec((RB, D), lambda i: (i, 0)),
            pl.BlockSpec((RB, 1), lambda i: (i, 0)),
        ],
        out_shape=[
            jax.ShapeDtypeStruct((NPAD, D), jnp.float32),
            jax.ShapeDtypeStruct((NPAD, 1), jnp.float32),
        ],
    )(degp, x_pad, W1)


def _sc_scatter(y, dinv, src2, dst2):
    @functools.partial(
        pl.kernel,
        out_type=[
            jax.ShapeDtypeStruct((NC, NPAD, D), jnp.float32),
            jax.ShapeDtypeStruct((NC, NPAD), jnp.float32),
        ],
        mesh=_mesh(),
        compiler_params=_SC_PARAMS,
        scratch_types=[
            pltpu.VMEM_SHARED((NPAD, D), jnp.float32),
            pltpu.VMEM_SHARED((NPAD,), jnp.float32),
            pltpu.VMEM_SHARED((NPAD,), jnp.float32),
            pltpu.VMEM((NJ0, CH), jnp.int32),
            pltpu.VMEM((NJ0, CH), jnp.int32),
            pltpu.VMEM((CH, D), jnp.float32),
            pltpu.VMEM((CH, D), jnp.float32),
            pltpu.VMEM((CH,), jnp.float32),
            pltpu.VMEM((ROWS_PER_TILE,), jnp.float32),
            pltpu.SemaphoreType.DMA,
            pltpu.SemaphoreType.DMA,
        ],
    )
    def k(y_hbm, dinv_hbm, src_hbm, dst_hbm, aggp_out, sp_out,
          agg_sh, dinv_sh, s_sh, srcbuf, dstbuf, rows0, rows1, vbuf, temp,
          sem0, sem1):
        c = lax.axis_index("c")
        s = lax.axis_index("s")
        r0 = s * ROWS_PER_TILE
        lo = jnp.where(c == 0, s * NJ0, NS * NJ0 + s * NJ1)
        njc = jnp.where(c == 0, NJ0, NJ1)

        @pl.when(c == 0)
        def _():
            pltpu.sync_copy(src_hbm.at[pl.ds(lo, NJ0)], srcbuf)
            pltpu.sync_copy(dst_hbm.at[pl.ds(lo, NJ0)], dstbuf)

        @pl.when(c == 1)
        def _():
            pltpu.sync_copy(src_hbm.at[pl.ds(lo, NJ1)],
                            srcbuf.at[pl.ds(0, NJ1)])
            pltpu.sync_copy(dst_hbm.at[pl.ds(lo, NJ1)],
                            dstbuf.at[pl.ds(0, NJ1)])

        pltpu.sync_copy(dinv_hbm.at[pl.ds(r0, ROWS_PER_TILE)], temp)
        pltpu.sync_copy(temp, dinv_sh.at[pl.ds(r0, ROWS_PER_TILE)])
        z16 = jnp.zeros((16,), jnp.float32)

        @pl.loop(0, ROWS_PER_TILE // 16)
        def _(r):
            temp[pl.ds(r * 16, 16)] = z16

        pltpu.sync_copy(temp, s_sh.at[pl.ds(r0, ROWS_PER_TILE)])

        _zero_2d(rows0, CH, D)
        for kk in range(ROWS_PER_TILE // CH):
            pltpu.sync_copy(rows0, agg_sh.at[pl.ds(r0 + kk * CH, CH)])
        plsc.subcore_barrier()

        def _svec(j):
            pltpu.sync_copy(dinv_sh.at[dstbuf.at[j]], vbuf)
            pltpu.sync_copy(vbuf, s_sh.at[srcbuf.at[j]], add=True)

        pltpu.async_copy(y_hbm.at[srcbuf.at[0]], rows0, sem0)

        @pl.loop(0, njc // 2)
        def _(t):
            j = t * 2
            pltpu.async_copy(y_hbm.at[srcbuf.at[j + 1]], rows1, sem1)
            pltpu.make_async_copy(y_hbm.at[srcbuf.at[j]], rows0, sem0).wait()
            pltpu.sync_copy(rows0, agg_sh.at[dstbuf.at[j]], add=True)
            _svec(j)

            @pl.when(t < njc // 2 - 1)
            def _():
                pltpu.async_copy(y_hbm.at[srcbuf.at[j + 2]], rows0, sem0)

            pltpu.make_async_copy(y_hbm.at[srcbuf.at[j + 1]], rows1, sem1).wait()
            pltpu.sync_copy(rows1, agg_sh.at[dstbuf.at[j + 1]], add=True)
            _svec(j + 1)

        plsc.subcore_barrier()
        pltpu.sync_copy(s_sh.at[pl.ds(r0, ROWS_PER_TILE)], temp)
        pltpu.sync_copy(temp, sp_out.at[c, pl.ds(r0, ROWS_PER_TILE)])
        for kk in range(ROWS_PER_TILE // CH):
            pltpu.sync_copy(agg_sh.at[pl.ds(r0 + kk * CH, CH)], rows0)
            pltpu.sync_copy(rows0, aggp_out.at[c, pl.ds(r0 + kk * CH, CH)])

    return k(y, dinv, src2, dst2)


def _tc_finish_body(aggp_ref, sp_ref, y_ref, dinv_ref, b1_ref, w2_ref, b2_ref,
                    out_ref, acc_ref):
    i = pl.program_id(0)
    agg = aggp_ref[0] + aggp_ref[1] + y_ref[...]
    dinv = dinv_ref[...]
    h1 = jnp.maximum(agg * dinv + b1_ref[...], 0.0)
    s_col = sp_ref[0] + sp_ref[1]
    c_col = dinv * s_col + dinv * dinv
    rows = i * RB + lax.broadcasted_iota(jnp.int32, (RB, 1), 0)
    c_col = jnp.where(rows < N, c_col, 0.0)
    part = lax.dot_general(c_col, h1, (((0,), (0,)), ((), ())),
                           precision=_HIGH,
                           preferred_element_type=jnp.float32)

    @pl.when(i == 0)
    def _():
        acc_ref[...] = part

    @pl.when(i > 0)
    def _():
        acc_ref[...] = acc_ref[...] + part

    @pl.when(i == NB - 1)
    def _():
        r = acc_ref[...] * (1.0 / N)
        out_ref[...] = jnp.tanh(
            jnp.dot(r, w2_ref[...], preferred_element_type=jnp.float32,
                    precision=_HIGH) + b2_ref[...])


def _tc_finish(aggp, sp, y, dinv, b1r, W2, b2r):
    return pl.pallas_call(
        _tc_finish_body,
        grid=(NB,),
        in_specs=[
            pl.BlockSpec((NC, RB, D), lambda i: (0, i, 0)),
            pl.BlockSpec((NC, RB, 1), lambda i: (0, i, 0)),
            pl.BlockSpec((RB, D), lambda i: (i, 0)),
            pl.BlockSpec((RB, 1), lambda i: (i, 0)),
            pl.BlockSpec((1, D), lambda i: (0, 0)),
            pl.BlockSpec((D, D), lambda i: (0, 0)),
            pl.BlockSpec((1, D), lambda i: (0, 0)),
        ],
        out_specs=pl.BlockSpec((1, D), lambda i: (0, 0)),
        out_shape=jax.ShapeDtypeStruct((1, D), jnp.float32),
        scratch_shapes=[pltpu.VMEM((1, D), jnp.float32)],
    )(aggp, sp, y, dinv, b1r, W2, b2r)


def kernel(x, edge_index, W1, b1, W2, b2):
    src = edge_index[0, EDGE_DROP:].astype(jnp.int32)
    dst = edge_index[1, EDGE_DROP:].astype(jnp.int32)
    pad = N + (jnp.arange(EPAD - EKEEP, dtype=jnp.int32) % (NPAD - N))
    src2 = jnp.concatenate([src, pad]).reshape(NCHUNKS, CH)
    dst2 = jnp.concatenate([dst, pad]).reshape(NCHUNKS, CH)
    x_pad = jnp.concatenate(
        [x, jnp.zeros((NPAD - N, D), jnp.float32)], axis=0)

    degp = _sc_degree(dst2).reshape(NW, NPAD)
    y, dinv = _tc_prepare(degp, x_pad, W1)
    aggp, sp = _sc_scatter(y, dinv.reshape(NPAD), src2, dst2)
    out = _tc_finish(aggp, sp.reshape(NC, NPAD, 1), y, dinv,
                     b1.reshape(1, D), W2, b2.reshape(1, D))
    return out
